# Initial kernel scaffold; baseline (speedup 1.0000x reference)
import jax
import jax.numpy as jnp
from jax.experimental import pallas as pl


def kernel(feature, xyz, neigh_idx, params):
    raise NotImplementedError("write your pallas kernel here")


# speedup vs baseline: 2.9555x; 2.9555x over previous
import functools

import jax
import jax.numpy as jnp
from jax.experimental import pallas as pl
from jax.experimental.pallas import tpu as pltpu
from jax.experimental.pallas import tpu_sc as plsc

_TN = 2000


_WT = 128


def _sc_gather(table, idx):
    S = idx.shape[0]
    info = plsc.get_sparse_core_info()
    nw = info.num_cores * info.num_subcores
    per_w = S // nw
    ch = 400
    assert S % nw == 0 and per_w % ch == 0 and per_w % 8 == 0 and ch % 8 == 0
    n_it = per_w // ch
    mesh = plsc.VectorSubcoreMesh(core_axis_name="c", subcore_axis_name="s")

    @functools.partial(
        pl.kernel,
        mesh=mesh,
        out_type=jax.ShapeDtypeStruct((S, _WT), jnp.float32),
        scratch_types=[
            pltpu.VMEM((ch,), jnp.int32),
            pltpu.VMEM((ch, _WT), jnp.float32),
            pltpu.SemaphoreType.DMA,
        ],
    )
    def gk(idx_hbm, table_hbm, out_hbm, idx_v, rows_v, sem):
        wid = jax.lax.axis_index("s") * info.num_cores + jax.lax.axis_index("c")
        base0 = wid * per_w

        def body(i, carry):
            base = base0 + i * ch
            pltpu.sync_copy(idx_hbm.at[pl.ds(base, ch)], idx_v)
            pltpu.async_copy(table_hbm.at[idx_v], rows_v, sem).wait()
            pltpu.sync_copy(rows_v, out_hbm.at[pl.ds(base, ch)])
            return carry

        jax.lax.fori_loop(0, n_it, body, 0)

    return gk(idx, table)


def _affine_from(p, mu, sig):
    hi = jax.lax.Precision.HIGHEST
    w = p["W"]
    mean_y = jnp.dot(w, mu.T, precision=hi)[:, 0]
    var_y = jnp.sum(jnp.dot(w, sig, precision=hi) * w, axis=1)
    s = p["g"] / jnp.sqrt(var_y + 1e-5)
    at = (w * s[:, None]).T
    c = (p["b"] - s * mean_y)[None, :]
    return at, c


def _bn_affine(p, sx, sxx, count):
    hi = jax.lax.Precision.HIGHEST
    mu = sx / count
    sig = sxx / count - jnp.dot(mu.T, mu, precision=hi)
    return _affine_from(p, mu, sig)


def _stats(x):
    r, c = x.shape
    t_n = r // _TN

    def kfn(x_ref, sx_ref, sxx_ref, accv, accm):
        t = pl.program_id(0)

        @pl.when(t == 0)
        def _init():
            accv[...] = jnp.zeros_like(accv)
            accm[...] = jnp.zeros_like(accm)

        xb = x_ref[...]
        accv[...] += jnp.sum(xb, axis=0, keepdims=True)
        accm[...] += jax.lax.dot_general(
            xb, xb, (((0,), (0,)), ((), ())),
            preferred_element_type=jnp.float32,
            precision=jax.lax.Precision.HIGHEST)

        @pl.when(t == t_n - 1)
        def _fin():
            sx_ref[...] = accv[...]
            sxx_ref[...] = accm[...]

    return pl.pallas_call(
        kfn,
        grid=(t_n,),
        in_specs=[pl.BlockSpec((_TN, c), lambda t: (t, 0))],
        out_specs=[pl.BlockSpec((1, c), lambda t: (0, 0)),
                   pl.BlockSpec((c, c), lambda t: (0, 0))],
        out_shape=[jax.ShapeDtypeStruct((1, c), jnp.float32),
                   jax.ShapeDtypeStruct((c, c), jnp.float32)],
        scratch_shapes=[pltpu.VMEM((1, c), jnp.float32),
                        pltpu.VMEM((c, c), jnp.float32)],
    )(x)


def _cstats(x, mu):
    r, c = x.shape
    t_n = r // _TN

    def kfn(x_ref, m_ref, sxx_ref, accm):
        t = pl.program_id(0)

        @pl.when(t == 0)
        def _init():
            accm[...] = jnp.zeros_like(accm)

        xb = x_ref[...] - m_ref[...]
        accm[...] += jax.lax.dot_general(
            xb, xb, (((0,), (0,)), ((), ())),
            preferred_element_type=jnp.float32,
            precision=jax.lax.Precision.HIGHEST)

        @pl.when(t == t_n - 1)
        def _fin():
            sxx_ref[...] = accm[...]

    return pl.pallas_call(
        kfn,
        grid=(t_n,),
        in_specs=[pl.BlockSpec((_TN, c), lambda t: (t, 0)),
                  pl.BlockSpec((1, c), lambda t: (0, 0))],
        out_specs=pl.BlockSpec((c, c), lambda t: (0, 0)),
        out_shape=jax.ShapeDtypeStruct((c, c), jnp.float32),
        scratch_shapes=[pltpu.VMEM((c, c), jnp.float32)],
    )(x, mu)


def _centered_moments(x):
    r = x.shape[0]
    sx, _ = _stats(x)
    mu = sx / r
    sig = _cstats(x, mu) / r
    return mu, sig


def _build_table(feat, xyz, a2t, c2):
    r, d_in = feat.shape
    dm = a2t.shape[1]
    d_tab = _WT
    pad = d_tab - 3 - dm
    t_n = r // _TN

    def kfn(f_ref, x_ref, a_ref, c_ref, o_ref):
        fpc = jnp.maximum(
            jnp.dot(f_ref[...], a_ref[...],
                    preferred_element_type=jnp.float32,
            precision=jax.lax.Precision.HIGHEST) + c_ref[...], 0.0)
        o_ref[...] = jnp.concatenate(
            [x_ref[...], fpc, jnp.zeros((_TN, pad), jnp.float32)], axis=1)

    return pl.pallas_call(
        kfn,
        grid=(t_n,),
        in_specs=[pl.BlockSpec((_TN, d_in), lambda t: (t, 0)),
                  pl.BlockSpec((_TN, 3), lambda t: (t, 0)),
                  pl.BlockSpec((d_in, dm), lambda t: (0, 0)),
                  pl.BlockSpec((1, dm), lambda t: (0, 0))],
        out_specs=pl.BlockSpec((_TN, d_tab), lambda t: (t, 0)),
        out_shape=jax.ShapeDtypeStruct((r, d_tab), jnp.float32),
    )(feat, xyz, a2t, c2)


def _enc(g, xyzb):
    nx = g[:, 0:3]
    rel = xyzb - nx
    dis = jnp.sqrt(jnp.sum(rel * rel, axis=1, keepdims=True))
    return jnp.concatenate([dis, rel, xyzb, nx], axis=1)


def _enc_stats(g3, xyz):
    k_n, r, d_tab = g3.shape
    t_n = r // _TN

    def kfn(g_ref, x_ref, sx_ref, sxx_ref, accv, accm):
        t = pl.program_id(0)
        k = pl.program_id(1)

        @pl.when(jnp.logical_and(t == 0, k == 0))
        def _init():
            accv[...] = jnp.zeros_like(accv)
            accm[...] = jnp.zeros_like(accm)

        enc = _enc(g_ref[0], x_ref[...])
        accv[...] += jnp.sum(enc, axis=0, keepdims=True)
        accm[...] += jax.lax.dot_general(
            enc, enc, (((0,), (0,)), ((), ())),
            preferred_element_type=jnp.float32,
            precision=jax.lax.Precision.HIGHEST)

        @pl.when(jnp.logical_and(t == t_n - 1, k == k_n - 1))
        def _fin():
            sx_ref[...] = accv[...]
            sxx_ref[...] = accm[...]

    return pl.pallas_call(
        kfn,
        grid=(t_n, k_n),
        in_specs=[pl.BlockSpec((1, _TN, d_tab), lambda t, k: (k, t, 0)),
                  pl.BlockSpec((_TN, 3), lambda t, k: (t, 0))],
        out_specs=[pl.BlockSpec((1, 10), lambda t, k: (0, 0)),
                   pl.BlockSpec((10, 10), lambda t, k: (0, 0))],
        out_shape=[jax.ShapeDtypeStruct((1, 10), jnp.float32),
                   jax.ShapeDtypeStruct((10, 10), jnp.float32)],
        scratch_shapes=[pltpu.VMEM((1, 10), jnp.float32),
                        pltpu.VMEM((10, 10), jnp.float32)],
    )(g3, xyz)


def _cc_stats(g3, xyz, a1t, c1, dm):
    k_n, r, d_tab = g3.shape
    t_n = r // _TN
    d2 = 2 * dm

    def kfn(g_ref, x_ref, a1_ref, c1_ref, sx_ref, sxx_ref, accv, accm):
        t = pl.program_id(0)
        k = pl.program_id(1)

        @pl.when(jnp.logical_and(t == 0, k == 0))
        def _init():
            accv[...] = jnp.zeros_like(accv)
            accm[...] = jnp.zeros_like(accm)

        g = g_ref[0]
        enc = _enc(g, x_ref[...])
        fxyz = jnp.maximum(
            jnp.dot(enc, a1_ref[...],
                    preferred_element_type=jnp.float32,
            precision=jax.lax.Precision.HIGHEST) + c1_ref[...], 0.0)
        cc = jnp.concatenate([g[:, 3:3 + dm], fxyz], axis=1)
        accv[...] += jnp.sum(cc, axis=0, keepdims=True)
        accm[...] += jax.lax.dot_general(
            cc, cc, (((0,), (0,)), ((), ())),
            preferred_element_type=jnp.float32,
            precision=jax.lax.Precision.HIGHEST)

        @pl.when(jnp.logical_and(t == t_n - 1, k == k_n - 1))
        def _fin():
            sx_ref[...] = accv[...]
            sxx_ref[...] = accm[...]

    return pl.pallas_call(
        kfn,
        grid=(t_n, k_n),
        in_specs=[pl.BlockSpec((1, _TN, d_tab), lambda t, k: (k, t, 0)),
                  pl.BlockSpec((_TN, 3), lambda t, k: (t, 0)),
                  pl.BlockSpec((10, dm), lambda t, k: (0, 0)),
                  pl.BlockSpec((1, dm), lambda t, k: (0, 0))],
        out_specs=[pl.BlockSpec((1, d2), lambda t, k: (0, 0)),
                   pl.BlockSpec((d2, d2), lambda t, k: (0, 0))],
        out_shape=[jax.ShapeDtypeStruct((1, d2), jnp.float32),
                   jax.ShapeDtypeStruct((d2, d2), jnp.float32)],
        scratch_shapes=[pltpu.VMEM((1, d2), jnp.float32),
                        pltpu.VMEM((d2, d2), jnp.float32)],
    )(g3, xyz, a1t, c1)


def _attention(g3, xyz, a1t, c1, a3t, c3, fcw, fcb, dm):
    k_n, r, d_tab = g3.shape
    t_n = r // _TN
    d2 = 2 * dm

    def kfn(g_ref, x_ref, a1_ref, c1_ref, a3_ref, c3_ref, fw_ref, fb_ref,
            fa_ref, sx_ref, sxx_ref, smax, ssum, accv, accm):
        t = pl.program_id(0)
        k = pl.program_id(1)

        @pl.when(jnp.logical_and(t == 0, k == 0))
        def _init():
            accv[...] = jnp.zeros_like(accv)
            accm[...] = jnp.zeros_like(accm)

        g = g_ref[0]
        enc = _enc(g, x_ref[...])
        fxyz = jnp.maximum(
            jnp.dot(enc, a1_ref[...],
                    preferred_element_type=jnp.float32,
            precision=jax.lax.Precision.HIGHEST) + c1_ref[...], 0.0)
        cc = jnp.concatenate([g[:, 3:3 + dm], fxyz], axis=1)
        f3 = jnp.maximum(
            jnp.dot(cc, a3_ref[...],
                    preferred_element_type=jnp.float32,
            precision=jax.lax.Precision.HIGHEST) + c3_ref[...], 0.0)
        wk = fw_ref[0, k]

        @pl.when(k == 0)
        def _first():
            smax[...] = f3
            ssum[...] = f3 * wk

        @pl.when(k > 0)
        def _rest():
            smax[...] = jnp.maximum(smax[...], f3)
            ssum[...] += f3 * wk

        @pl.when(k == k_n - 1)
        def _pool():
            score = ssum[...] + fb_ref[0, 0]
            m = jnp.max(score, axis=1, keepdims=True)
            e = jnp.exp(score - m)
            soft = e / jnp.sum(e, axis=1, keepdims=True)
            fa = smax[...] * (1.0 + soft)
            fa_ref[...] = fa
            accv[...] += jnp.sum(fa, axis=0, keepdims=True)
            accm[...] += jax.lax.dot_general(
                fa, fa, (((0,), (0,)), ((), ())),
                preferred_element_type=jnp.float32,
            precision=jax.lax.Precision.HIGHEST)

        @pl.when(jnp.logical_and(t == t_n - 1, k == k_n - 1))
        def _fin():
            sx_ref[...] = accv[...]
            sxx_ref[...] = accm[...]

    return pl.pallas_call(
        kfn,
        grid=(t_n, k_n),
        in_specs=[pl.BlockSpec((1, _TN, d_tab), lambda t, k: (k, t, 0)),
                  pl.BlockSpec((_TN, 3), lambda t, k: (t, 0)),
                  pl.BlockSpec((10, dm), lambda t, k: (0, 0)),
                  pl.BlockSpec((1, dm), lambda t, k: (0, 0)),
                  pl.BlockSpec((d2, dm), lambda t, k: (0, 0)),
                  pl.BlockSpec((1, dm), lambda t, k: (0, 0)),
                  pl.BlockSpec(memory_space=pltpu.SMEM),
                  pl.BlockSpec(memory_space=pltpu.SMEM)],
        out_specs=[pl.BlockSpec((_TN, dm), lambda t, k: (t, 0)),
                   pl.BlockSpec((1, dm), lambda t, k: (0, 0)),
                   pl.BlockSpec((dm, dm), lambda t, k: (0, 0))],
        out_shape=[jax.ShapeDtypeStruct((r, dm), jnp.float32),
                   jax.ShapeDtypeStruct((1, dm), jnp.float32),
                   jax.ShapeDtypeStruct((dm, dm), jnp.float32)],
        scratch_shapes=[pltpu.VMEM((_TN, dm), jnp.float32),
                        pltpu.VMEM((_TN, dm), jnp.float32),
                        pltpu.VMEM((1, dm), jnp.float32),
                        pltpu.VMEM((dm, dm), jnp.float32)],
    )(g3, xyz, a1t, c1, a3t, c3, fcw, fcb)


def _final(fa, feat, a4t, c4, asct, csc, d_out):
    r, dm = fa.shape
    d_in = feat.shape[1]
    t_n = r // _TN

    def kfn(fa_ref, f_ref, a4_ref, c4_ref, as_ref, cs_ref,
            o_ref, sx_ref, sxx_ref, accv, accm):
        t = pl.program_id(0)

        @pl.when(t == 0)
        def _init():
            accv[...] = jnp.zeros_like(accv)
            accm[...] = jnp.zeros_like(accm)

        y = jnp.maximum(
            jnp.dot(fa_ref[...], a4_ref[...],
                    preferred_element_type=jnp.float32,
            precision=jax.lax.Precision.HIGHEST) + c4_ref[...], 0.0)
        y += jnp.maximum(
            jnp.dot(f_ref[...], as_ref[...],
                    preferred_element_type=jnp.float32,
            precision=jax.lax.Precision.HIGHEST) + cs_ref[...], 0.0)
        o_ref[...] = y
        accv[...] += jnp.sum(y, axis=0, keepdims=True)
        accm[...] += jax.lax.dot_general(
            y, y, (((0,), (0,)), ((), ())),
            preferred_element_type=jnp.float32,
            precision=jax.lax.Precision.HIGHEST)

        @pl.when(t == t_n - 1)
        def _fin():
            sx_ref[...] = accv[...]
            sxx_ref[...] = accm[...]

    return pl.pallas_call(
        kfn,
        grid=(t_n,),
        in_specs=[pl.BlockSpec((_TN, dm), lambda t: (t, 0)),
                  pl.BlockSpec((_TN, d_in), lambda t: (t, 0)),
                  pl.BlockSpec((dm, d_out), lambda t: (0, 0)),
                  pl.BlockSpec((1, d_out), lambda t: (0, 0)),
                  pl.BlockSpec((d_in, d_out), lambda t: (0, 0)),
                  pl.BlockSpec((1, d_out), lambda t: (0, 0))],
        out_specs=[pl.BlockSpec((_TN, d_out), lambda t: (t, 0)),
                   pl.BlockSpec((1, d_out), lambda t: (0, 0)),
                   pl.BlockSpec((d_out, d_out), lambda t: (0, 0))],
        out_shape=[jax.ShapeDtypeStruct((r, d_out), jnp.float32),
                   jax.ShapeDtypeStruct((1, d_out), jnp.float32),
                   jax.ShapeDtypeStruct((d_out, d_out), jnp.float32)],
        scratch_shapes=[pltpu.VMEM((1, d_out), jnp.float32),
                        pltpu.VMEM((d_out, d_out), jnp.float32)],
    )(fa, feat, a4t, c4, asct, csc)


def _block(p, feat, xyz, idx_flat, dm, d_out):
    bn = feat.shape[0]
    s = idx_flat.shape[0]
    fmu, fsig = _centered_moments(feat)
    a2t, c2 = _affine_from(p["mlp2"], fmu, fsig)
    asct, csc = _affine_from(p["shortcut"], fmu, fsig)
    table = _build_table(feat, xyz, a2t, c2)
    g = _sc_gather(table, idx_flat)
    g3 = g.reshape(s // bn, bn, _WT)
    esx, esxx = _enc_stats(g3, xyz)
    a1t, c1 = _bn_affine(p["mlp1"], esx, esxx, s)
    csx, csxx = _cc_stats(g3, xyz, a1t, c1, dm)
    a3t, c3 = _bn_affine(p["mlp3"], csx, csxx, s)
    fcw = p["fc_w"].reshape(1, -1)
    fcb = p["fc_b"].reshape(1, 1)
    fa, _, _ = _attention(g3, xyz, a1t, c1, a3t, c3, fcw, fcb, dm)
    amu, asig = _centered_moments(fa)
    a4t, c4 = _affine_from(p["mlp4"], amu, asig)
    out, _, _ = _final(fa, feat, a4t, c4, asct, csc, d_out)
    return out


def kernel(feature, xyz, neigh_idx, params):
    b, d_in, n = feature.shape[0], feature.shape[1], feature.shape[2]
    k_n = neigh_idx.shape[-1]
    bn = b * n
    feat = feature[..., 0].transpose(0, 2, 1).reshape(bn, d_in)
    xyzf = xyz.reshape(bn, 3)
    offs = (jnp.arange(b, dtype=jnp.int32) * n)[:, None, None]
    idxg = neigh_idx.astype(jnp.int32) + offs
    idx_flat = jnp.transpose(idxg, (2, 0, 1)).reshape(k_n * bn)
    out1 = _block(params["block1"], feat, xyzf, idx_flat, dm=8, d_out=32)
    out2 = _block(params["block2"], out1, xyzf, idx_flat, dm=16, d_out=64)
    d_out = out2.shape[1]
    return out2.reshape(b, n, d_out).transpose(0, 2, 1)[..., None]
